# baseline (device time: 21058 ns/iter reference)
import jax
import jax.numpy as jnp
from jax import lax
from jax.experimental import pallas as pl
from jax.experimental.pallas import tpu as pltpu

N_DEV = 4
B, SQ, SKV, DH = 2, 256, 256, 64
D_MODEL = 512
H_PER = 4

_BF = jnp.bfloat16


def kernel(x, Wq, K_ext, V_ext, Wo):
    K_t = K_ext.reshape(B, SKV, 16 * DH)
    V_t = V_ext.reshape(B, SKV, 16 * DH)
    x2 = x.reshape(B * SQ, D_MODEL)

    def body(x_ref, wq_ref, k_ref, v_ref, wo_ref, out_ref,
             comm_wq, comm_wo, wq_ssem, wq_rsem, wo_ssem, wo_rsem):
        my = lax.axis_index("i")
        left = (my - 1) % N_DEV
        right = (my + 1) % N_DEV
        diag = (my + 2) % N_DEV

        barrier = pltpu.get_barrier_semaphore()
        for nbr in (left, right, diag):
            pl.semaphore_signal(
                barrier, inc=1,
                device_id=(nbr,), device_id_type=pl.DeviceIdType.MESH,
            )
        pl.semaphore_wait(barrier, 3)

        def one_copy(comm, ssem, rsem, dst_slot, target):
            return pltpu.make_async_remote_copy(
                src_ref=comm.at[3],
                dst_ref=comm.at[dst_slot],
                send_sem=ssem.at[dst_slot],
                recv_sem=rsem.at[dst_slot],
                device_id=(target,),
                device_id_type=pl.DeviceIdType.MESH,
            )

        dests = ((1, right), (0, left), (2, diag))

        comm_wq[3, :, :] = wq_ref[...].astype(_BF)
        wq_sends = [one_copy(comm_wq, wq_ssem, wq_rsem, s, t) for s, t in dests]
        for rd in wq_sends:
            rd.start()
        comm_wo[3, :, :] = wo_ref[...].astype(_BF)
        wo_sends = [one_copy(comm_wo, wo_ssem, wo_rsem, s, t) for s, t in dests]
        for rd in wo_sends:
            rd.start()
        sends = wq_sends + wo_sends

        x_bf = (x_ref[...] * 0.125).astype(_BF)
        qblk = lax.broadcasted_iota(jnp.int32, (SQ, SKV), 0) // 64
        kblk = lax.broadcasted_iota(jnp.int32, (SQ, SKV), 1) // 64
        maskf = jnp.where(qblk == kblk, 1.0, 0.0).astype(_BF)

        def contrib_attn(wq_j, origin):
            q2 = jnp.dot(
                x_bf, wq_j, preferred_element_type=jnp.float32
            ).astype(_BF)
            out = []
            for b in range(B):
                qb = q2[b * SQ:(b + 1) * SQ]
                k4 = k_ref[b, :, pl.ds(origin * (H_PER * DH), H_PER * DH)]
                v4 = v_ref[b, :, pl.ds(origin * (H_PER * DH), H_PER * DH)]
                k4 = k4.astype(_BF)
                v4 = v4.astype(_BF)
                ctxs = []
                for hh in range(H_PER):
                    qh = qb[:, hh * DH:(hh + 1) * DH]
                    s = lax.dot_general(
                        qh, k4[:, hh * DH:(hh + 1) * DH],
                        (((1,), (1,)), ((), ())),
                        preferred_element_type=jnp.float32,
                    ).astype(_BF)
                    w = jnp.exp(s) * maskf
                    denom = jnp.sum(
                        w, axis=-1, keepdims=True, dtype=jnp.float32
                    )
                    ctx = jnp.dot(
                        w, v4[:, hh * DH:(hh + 1) * DH],
                        preferred_element_type=jnp.float32,
                    )
                    ctxs.append((ctx / denom).astype(_BF))
                out.append(jnp.concatenate(ctxs, axis=1))
            return out

        def contrib_out(ctx_cats, wo_j, accs):
            return [
                acc + jnp.dot(cc, wo_j, preferred_element_type=jnp.float32)
                for acc, cc in zip(accs, ctx_cats)
            ]

        accs = [jnp.zeros((SQ, D_MODEL), jnp.float32) for _ in range(B)]

        accs = contrib_out(contrib_attn(comm_wq[3], my), comm_wo[3], accs)

        for (slot, _), origin in zip(dests, (left, right, diag)):
            one_copy(comm_wq, wq_ssem, wq_rsem, slot, right).wait_recv()
            ctx_cats = contrib_attn(comm_wq[slot], origin)
            one_copy(comm_wo, wo_ssem, wo_rsem, slot, right).wait_recv()
            accs = contrib_out(ctx_cats, comm_wo[slot], accs)

        for rd in sends:
            rd.wait_send()

        for b in range(B):
            out_ref[b] = accs[b]

    out_shape = jax.ShapeDtypeStruct((B, SQ, D_MODEL), jnp.float32)
    return pl.pallas_call(
        body,
        out_shape=out_shape,
        in_specs=[pl.BlockSpec(memory_space=pltpu.VMEM)] * 5,
        out_specs=pl.BlockSpec(memory_space=pltpu.VMEM),
        scratch_shapes=[
            pltpu.VMEM((N_DEV, D_MODEL, SQ), _BF),
            pltpu.VMEM((N_DEV, SQ, D_MODEL), _BF),
            pltpu.SemaphoreType.DMA((3,)),
            pltpu.SemaphoreType.DMA((3,)),
            pltpu.SemaphoreType.DMA((3,)),
            pltpu.SemaphoreType.DMA((3,)),
        ],
        compiler_params=pltpu.CompilerParams(collective_id=0),
    )(x2, Wq, K_t, V_t, Wo)


# device time: 19278 ns/iter; 1.0923x vs baseline; 1.0923x over previous
import jax
import jax.numpy as jnp
from jax import lax
from jax.experimental import pallas as pl
from jax.experimental.pallas import tpu as pltpu

N_DEV = 4
B, SQ, SKV, DH = 2, 256, 256, 64
D_MODEL = 512
H_PER = 4

_BF = jnp.bfloat16
_F8 = jnp.float8_e4m3fn
_WQ_SCALE = 32.0


def kernel(x, Wq, K_ext, V_ext, Wo):
    K_t = K_ext.reshape(B, SKV, 16 * DH)
    V_t = V_ext.reshape(B, SKV, 16 * DH)
    x2 = x.reshape(B * SQ, D_MODEL)

    def body(x_ref, wq_ref, k_ref, v_ref, wo_ref, out_ref,
             comm_wq, comm_wo, wq_ssem, wq_rsem, wo_ssem, wo_rsem):
        my = lax.axis_index("i")
        left = (my - 1) % N_DEV
        right = (my + 1) % N_DEV
        diag = (my + 2) % N_DEV

        barrier = pltpu.get_barrier_semaphore()
        for nbr in (left, right, diag):
            pl.semaphore_signal(
                barrier, inc=1,
                device_id=(nbr,), device_id_type=pl.DeviceIdType.MESH,
            )
        comm_wq[3, :, :] = (wq_ref[...] * _WQ_SCALE).astype(_F8)
        comm_wo[3, :, :] = wo_ref[...].astype(_BF)
        pl.semaphore_wait(barrier, 3)

        def one_copy(comm, ssem, rsem, dst_slot, target):
            return pltpu.make_async_remote_copy(
                src_ref=comm.at[3],
                dst_ref=comm.at[dst_slot],
                send_sem=ssem.at[dst_slot],
                recv_sem=rsem.at[dst_slot],
                device_id=(target,),
                device_id_type=pl.DeviceIdType.MESH,
            )

        dests = ((1, right), (0, left), (2, diag))

        wq_sends = [one_copy(comm_wq, wq_ssem, wq_rsem, s, t) for s, t in dests]
        for rd in wq_sends:
            rd.start()
        wo_sends = [one_copy(comm_wo, wo_ssem, wo_rsem, s, t) for s, t in dests]
        for rd in wo_sends:
            rd.start()
        sends = wq_sends + wo_sends

        x_bf = (x_ref[...] * (0.125 / _WQ_SCALE)).astype(_BF)
        qblk = lax.broadcasted_iota(jnp.int32, (SQ, SKV), 0) // 64
        kblk = lax.broadcasted_iota(jnp.int32, (SQ, SKV), 1) // 64
        maskf = jnp.where(qblk == kblk, 1.0, 0.0).astype(_BF)

        def contrib_attn(wq_j, origin):
            q2 = jnp.dot(
                x_bf, wq_j.astype(_BF), preferred_element_type=jnp.float32
            ).astype(_BF)
            out = []
            for b in range(B):
                qb = q2[b * SQ:(b + 1) * SQ]
                k4 = k_ref[b, :, pl.ds(origin * (H_PER * DH), H_PER * DH)]
                v4 = v_ref[b, :, pl.ds(origin * (H_PER * DH), H_PER * DH)]
                k4 = k4.astype(_BF)
                v4 = v4.astype(_BF)
                ctxs = []
                for hh in range(H_PER):
                    qh = qb[:, hh * DH:(hh + 1) * DH]
                    s = lax.dot_general(
                        qh, k4[:, hh * DH:(hh + 1) * DH],
                        (((1,), (1,)), ((), ())),
                        preferred_element_type=jnp.float32,
                    ).astype(_BF)
                    w = jnp.exp(s) * maskf
                    denom = jnp.sum(
                        w, axis=-1, keepdims=True, dtype=jnp.float32
                    )
                    ctx = jnp.dot(
                        w, v4[:, hh * DH:(hh + 1) * DH],
                        preferred_element_type=jnp.float32,
                    )
                    ctxs.append((ctx / denom).astype(_BF))
                out.append(jnp.concatenate(ctxs, axis=1))
            return out

        def contrib_out(ctx_cats, wo_j, accs):
            return [
                acc + jnp.dot(cc, wo_j, preferred_element_type=jnp.float32)
                for acc, cc in zip(accs, ctx_cats)
            ]

        accs = [jnp.zeros((SQ, D_MODEL), jnp.float32) for _ in range(B)]

        accs = contrib_out(contrib_attn(comm_wq[3], my), comm_wo[3], accs)

        for (slot, _), origin in zip(dests, (left, right, diag)):
            one_copy(comm_wq, wq_ssem, wq_rsem, slot, right).wait_recv()
            ctx_cats = contrib_attn(comm_wq[slot], origin)
            one_copy(comm_wo, wo_ssem, wo_rsem, slot, right).wait_recv()
            accs = contrib_out(ctx_cats, comm_wo[slot], accs)

        for rd in sends:
            rd.wait_send()

        for b in range(B):
            out_ref[b] = accs[b]

    out_shape = jax.ShapeDtypeStruct((B, SQ, D_MODEL), jnp.float32)
    return pl.pallas_call(
        body,
        out_shape=out_shape,
        in_specs=[pl.BlockSpec(memory_space=pltpu.VMEM)] * 5,
        out_specs=pl.BlockSpec(memory_space=pltpu.VMEM),
        scratch_shapes=[
            pltpu.VMEM((N_DEV, D_MODEL, SQ), _F8),
            pltpu.VMEM((N_DEV, SQ, D_MODEL), _BF),
            pltpu.SemaphoreType.DMA((3,)),
            pltpu.SemaphoreType.DMA((3,)),
            pltpu.SemaphoreType.DMA((3,)),
            pltpu.SemaphoreType.DMA((3,)),
        ],
        compiler_params=pltpu.CompilerParams(collective_id=0),
    )(x2, Wq, K_t, V_t, Wo)


# device time: 19267 ns/iter; 1.0930x vs baseline; 1.0006x over previous
import jax
import jax.numpy as jnp
from jax import lax
from jax.experimental import pallas as pl
from jax.experimental.pallas import tpu as pltpu

N_DEV = 4
B, SQ, SKV, DH = 2, 256, 256, 64
D_MODEL = 512
H_PER = 4

_BF = jnp.bfloat16
_F8 = jnp.float8_e4m3fn
_WQ_SCALE = 32.0


def kernel(x, Wq, K_ext, V_ext, Wo):
    K_t = K_ext.reshape(B, SKV, 16 * DH)
    V_t = V_ext.reshape(B, SKV, 16 * DH)
    x2 = x.reshape(B * SQ, D_MODEL)

    def body(x_ref, wq_ref, k_ref, v_ref, wo_ref, out_ref,
             comm_wq, comm_wo, comm_s,
             wq_ssem, wq_rsem, wo_ssem, wo_rsem, s_ssem, s_rsem):
        my = lax.axis_index("i")
        left = (my - 1) % N_DEV
        right = (my + 1) % N_DEV
        diag = (my + 2) % N_DEV

        barrier = pltpu.get_barrier_semaphore()
        for nbr in (left, right, diag):
            pl.semaphore_signal(
                barrier, inc=1,
                device_id=(nbr,), device_id_type=pl.DeviceIdType.MESH,
            )
        comm_wq[3, :, :] = (wq_ref[...] * _WQ_SCALE).astype(_F8)
        wo_f = wo_ref[...]
        s_row = jnp.max(jnp.abs(wo_f), axis=1, keepdims=True)
        comm_wo[3, :, :] = jnp.floor(
            wo_f * (127.0 / s_row) + 0.5
        ).astype(jnp.int8)
        comm_s[3, 0, :] = (s_row[:, 0] * (1.0 / 127.0)).astype(jnp.float32)
        pl.semaphore_wait(barrier, 3)

        def one_copy(comm, ssem, rsem, dst_slot, target):
            return pltpu.make_async_remote_copy(
                src_ref=comm.at[3],
                dst_ref=comm.at[dst_slot],
                send_sem=ssem.at[dst_slot],
                recv_sem=rsem.at[dst_slot],
                device_id=(target,),
                device_id_type=pl.DeviceIdType.MESH,
            )

        dests = ((1, right), (0, left), (2, diag))

        wq_sends = [one_copy(comm_wq, wq_ssem, wq_rsem, s, t) for s, t in dests]
        for rd in wq_sends:
            rd.start()
        wo_sends = [one_copy(comm_wo, wo_ssem, wo_rsem, s, t) for s, t in dests]
        for rd in wo_sends:
            rd.start()
        s_sends = [one_copy(comm_s, s_ssem, s_rsem, s, t) for s, t in dests]
        for rd in s_sends:
            rd.start()
        sends = wq_sends + wo_sends + s_sends

        x_bf = (x_ref[...] * (0.125 / _WQ_SCALE)).astype(_BF)
        qblk = lax.broadcasted_iota(jnp.int32, (SQ, SKV), 0) // 64
        kblk = lax.broadcasted_iota(jnp.int32, (SQ, SKV), 1) // 64
        maskf = jnp.where(qblk == kblk, 1.0, 0.0).astype(_BF)

        def contrib_attn(wq_j, origin):
            q2 = jnp.dot(
                x_bf, wq_j.astype(_BF), preferred_element_type=jnp.float32
            ).astype(_BF)
            out = []
            for b in range(B):
                qb = q2[b * SQ:(b + 1) * SQ]
                k4 = k_ref[b, :, pl.ds(origin * (H_PER * DH), H_PER * DH)]
                v4 = v_ref[b, :, pl.ds(origin * (H_PER * DH), H_PER * DH)]
                k4 = k4.astype(_BF)
                v4 = v4.astype(_BF)
                ctxs = []
                for hh in range(H_PER):
                    qh = qb[:, hh * DH:(hh + 1) * DH]
                    s = lax.dot_general(
                        qh, k4[:, hh * DH:(hh + 1) * DH],
                        (((1,), (1,)), ((), ())),
                        preferred_element_type=jnp.float32,
                    ).astype(_BF)
                    w = jnp.exp(s) * maskf
                    denom = jnp.sum(
                        w, axis=-1, keepdims=True, dtype=jnp.float32
                    )
                    ctx = jnp.dot(
                        w, v4[:, hh * DH:(hh + 1) * DH],
                        preferred_element_type=jnp.float32,
                    )
                    ctxs.append((ctx / denom).astype(_BF))
                out.append(jnp.concatenate(ctxs, axis=1))
            return out

        def contrib_out(ctx_cats, slot, accs):
            sv = comm_s[slot, 0, :].astype(_BF)
            wo_q = comm_wo[slot].astype(_BF)
            return [
                acc + jnp.dot(
                    cc * sv[None, :], wo_q,
                    preferred_element_type=jnp.float32,
                )
                for acc, cc in zip(accs, ctx_cats)
            ]

        accs = [jnp.zeros((SQ, D_MODEL), jnp.float32) for _ in range(B)]

        accs = contrib_out(contrib_attn(comm_wq[3], my), 3, accs)

        for (slot, _), origin in zip(dests, (left, right, diag)):
            one_copy(comm_wq, wq_ssem, wq_rsem, slot, right).wait_recv()
            ctx_cats = contrib_attn(comm_wq[slot], origin)
            one_copy(comm_wo, wo_ssem, wo_rsem, slot, right).wait_recv()
            one_copy(comm_s, s_ssem, s_rsem, slot, right).wait_recv()
            accs = contrib_out(ctx_cats, slot, accs)

        for rd in sends:
            rd.wait_send()

        for b in range(B):
            out_ref[b] = accs[b]

    out_shape = jax.ShapeDtypeStruct((B, SQ, D_MODEL), jnp.float32)
    return pl.pallas_call(
        body,
        out_shape=out_shape,
        in_specs=[pl.BlockSpec(memory_space=pltpu.VMEM)] * 5,
        out_specs=pl.BlockSpec(memory_space=pltpu.VMEM),
        scratch_shapes=[
            pltpu.VMEM((N_DEV, D_MODEL, SQ), _F8),
            pltpu.VMEM((N_DEV, SQ, D_MODEL), jnp.int8),
            pltpu.VMEM((N_DEV, 1, SQ), jnp.float32),
            pltpu.SemaphoreType.DMA((3,)),
            pltpu.SemaphoreType.DMA((3,)),
            pltpu.SemaphoreType.DMA((3,)),
            pltpu.SemaphoreType.DMA((3,)),
            pltpu.SemaphoreType.DMA((3,)),
            pltpu.SemaphoreType.DMA((3,)),
        ],
        compiler_params=pltpu.CompilerParams(collective_id=0),
    )(x2, Wq, K_t, V_t, Wo)
